# baseline (speedup 1.0000x reference)
import functools

import jax
import jax.numpy as jnp
from jax import lax
from jax.experimental import pallas as pl
from jax.experimental.pallas import tpu as pltpu
from jax.experimental.pallas import tpu_sc as plsc

E = 320000
T = 160000
N3 = 3 * T
H = 64

NC = 2
NS = 16
NW = NC * NS
CH = 128
NCH = 118
NP = NW * NCH * CH
ESH = E // NS
EW = E // NW


@functools.lru_cache(maxsize=None)
def _mesh():
    return plsc.VectorSubcoreMesh(
        core_axis_name="c", subcore_axis_name="s", num_cores=NC, num_subcores=NS)


def _zero_vmem(z_v, n):
    @pl.loop(0, n // 16)
    def _(k):
        z_v[pl.ds(k * 16, 16)] = jnp.zeros((16,), jnp.float32)


def _gather2_body(idx_hbm, ta_hbm, tb_hbm, outa_hbm, outb_hbm,
                  idx_v, a_v, b_v, s_v, ta_sh, tb_sh, sem_i, sem_a, sem_b):
    cid = lax.axis_index("c")
    sid = lax.axis_index("s")
    wid = sid * NC + cid
    sl = pl.ds(sid * ESH, ESH)

    idx_cp = pltpu.async_copy(idx_hbm.at[wid], idx_v, sem_i)
    pltpu.async_copy(ta_hbm.at[sl], s_v, sem_a).wait()
    pltpu.async_copy(s_v, ta_sh.at[sl], sem_a).wait()
    pltpu.async_copy(tb_hbm.at[sl], s_v, sem_b).wait()
    pltpu.async_copy(s_v, tb_sh.at[sl], sem_b).wait()
    idx_cp.wait()
    plsc.subcore_barrier()

    def start(j):
        pltpu.async_copy(ta_sh.at[idx_v.at[j]], a_v.at[j], sem_a)
        pltpu.async_copy(tb_sh.at[idx_v.at[j]], b_v.at[j], sem_b)

    def drain(j):
        pltpu.make_async_copy(ta_sh.at[idx_v.at[j]], a_v.at[j], sem_a).wait()
        pltpu.make_async_copy(tb_sh.at[idx_v.at[j]], b_v.at[j], sem_b).wait()

    start(0)

    @pl.loop(0, NCH - 1)
    def _(j):
        start(j + 1)
        drain(j)

    drain(NCH - 1)
    pltpu.async_copy(a_v, outa_hbm.at[wid], sem_a).wait()
    pltpu.async_copy(b_v, outb_hbm.at[wid], sem_b).wait()


@functools.lru_cache(maxsize=None)
def _gather2_kernel():
    return pl.kernel(
        _gather2_body,
        out_type=[jax.ShapeDtypeStruct((NW, NCH, CH), jnp.float32)] * 2,
        mesh=_mesh(),
        scratch_types=[
            pltpu.VMEM((NCH, CH), jnp.int32),
            pltpu.VMEM((NCH, CH), jnp.float32),
            pltpu.VMEM((NCH, CH), jnp.float32),
            pltpu.VMEM((ESH,), jnp.float32),
            pltpu.VMEM_SHARED((E,), jnp.float32),
            pltpu.VMEM_SHARED((E,), jnp.float32),
            pltpu.SemaphoreType.DMA,
            pltpu.SemaphoreType.DMA,
            pltpu.SemaphoreType.DMA,
        ],
    )


def _gather2(idx3, ta, tb):
    return _gather2_kernel()(idx3, ta, tb)


def _scatter_body(idx_hbm, val_hbm, out_hbm, idx_v, val_v, z_v, acc_sh, sem):
    cid = lax.axis_index("c")
    sid = lax.axis_index("s")
    wid = sid * NC + cid

    _zero_vmem(z_v, ESH)
    pltpu.async_copy(z_v, acc_sh.at[pl.ds(sid * ESH, ESH)], sem).wait()
    pltpu.async_copy(idx_hbm.at[wid], idx_v, sem).wait()
    pltpu.async_copy(val_hbm.at[wid], val_v, sem).wait()
    plsc.subcore_barrier()

    @pl.loop(0, NCH)
    def _(j):
        pltpu.sync_copy(val_v.at[j], acc_sh.at[idx_v.at[j]], add=True)

    plsc.subcore_barrier()
    pltpu.async_copy(acc_sh.at[pl.ds(sid * ESH, ESH)], z_v, sem).wait()
    pltpu.async_copy(
        z_v, out_hbm.at[pl.ds(cid * E + sid * ESH, ESH)], sem).wait()


@functools.lru_cache(maxsize=None)
def _scatter_add_kernel():
    return pl.kernel(
        _scatter_body,
        out_type=jax.ShapeDtypeStruct((NC * E,), jnp.float32),
        mesh=_mesh(),
        scratch_types=[
            pltpu.VMEM((NCH, CH), jnp.int32),
            pltpu.VMEM((NCH, CH), jnp.float32),
            pltpu.VMEM((ESH,), jnp.float32),
            pltpu.VMEM_SHARED((E,), jnp.float32),
            pltpu.SemaphoreType.DMA,
        ],
    )


def _scatter_combine_body(idx_hbm, val_hbm, ec_hbm, cnt_hbm, out_hbm,
                          idx_v, val_v, z_v, ec_v, cnt_v, acc_sh, sem):
    cid = lax.axis_index("c")
    sid = lax.axis_index("s")
    wid = sid * NC + cid

    _zero_vmem(z_v, ESH)
    pltpu.async_copy(z_v, acc_sh.at[pl.ds(sid * ESH, ESH)], sem).wait()
    plsc.subcore_barrier()

    def scatter_slab(slab):
        pltpu.async_copy(idx_hbm.at[slab], idx_v, sem).wait()
        pltpu.async_copy(val_hbm.at[slab], val_v, sem).wait()

        @pl.loop(0, NCH)
        def _(j):
            pltpu.sync_copy(val_v.at[j], acc_sh.at[idx_v.at[j]], add=True)

    scatter_slab(2 * sid)
    scatter_slab(2 * sid + 1)
    plsc.subcore_barrier()

    sl = pl.ds(wid * EW, EW)
    cp_e = pltpu.async_copy(ec_hbm.at[sl], ec_v, sem)
    cp_c = pltpu.async_copy(cnt_hbm.at[sl], cnt_v, sem)
    pltpu.async_copy(acc_sh.at[sl], z_v.at[pl.ds(0, EW)], sem).wait()
    cp_e.wait()
    cp_c.wait()

    @pl.loop(0, EW // 16)
    def _(k):
        s16 = pl.ds(k * 16, 16)
        ec = ec_v[s16]
        cnt = cnt_v[s16]
        eu = z_v[s16]
        ec_v[s16] = jnp.where(cnt > 0, 0.0, ec) + eu

    pltpu.async_copy(ec_v, out_hbm.at[sl], sem).wait()


@functools.lru_cache(maxsize=None)
def _scatter_combine_kernel():
    return pl.kernel(
        _scatter_combine_body,
        out_type=jax.ShapeDtypeStruct((E,), jnp.float32),
        mesh=_mesh(),
        scratch_types=[
            pltpu.VMEM((NCH, CH), jnp.int32),
            pltpu.VMEM((NCH, CH), jnp.float32),
            pltpu.VMEM((ESH,), jnp.float32),
            pltpu.VMEM((EW,), jnp.float32),
            pltpu.VMEM((EW,), jnp.int32),
            pltpu.VMEM_SHARED((E,), jnp.float32),
            pltpu.SemaphoreType.DMA,
        ],
    )


def _combine_sc_body(p_hbm, ec_hbm, cnt_hbm, out_hbm,
                     p0_v, p1_v, ec_v, cnt_v, sem):
    cid = lax.axis_index("c")
    sid = lax.axis_index("s")
    wid = sid * NC + cid
    sl = pl.ds(wid * EW, EW)

    cp0 = pltpu.async_copy(p_hbm.at[pl.ds(wid * EW, EW)], p0_v, sem)
    cp1 = pltpu.async_copy(p_hbm.at[pl.ds(E + wid * EW, EW)], p1_v, sem)
    cpe = pltpu.async_copy(ec_hbm.at[sl], ec_v, sem)
    cpc = pltpu.async_copy(cnt_hbm.at[sl], cnt_v, sem)
    cp0.wait()
    cp1.wait()
    cpe.wait()
    cpc.wait()

    @pl.loop(0, EW // 16)
    def _(k):
        s16 = pl.ds(k * 16, 16)
        ec_v[s16] = (jnp.where(cnt_v[s16] > 0, 0.0, ec_v[s16])
                     + p0_v[s16] + p1_v[s16])

    pltpu.async_copy(ec_v, out_hbm.at[sl], sem).wait()


@functools.lru_cache(maxsize=None)
def _combine_sc_kernel():
    return pl.kernel(
        _combine_sc_body,
        out_type=jax.ShapeDtypeStruct((E,), jnp.float32),
        mesh=_mesh(),
        scratch_types=[
            pltpu.VMEM((EW,), jnp.float32),
            pltpu.VMEM((EW,), jnp.float32),
            pltpu.VMEM((EW,), jnp.float32),
            pltpu.VMEM((EW,), jnp.int32),
            pltpu.SemaphoreType.DMA,
        ],
    )


def _mlp_t(h, w1g, b1, w2g, b2):
    ones_row = jnp.ones((1, H), jnp.bfloat16)
    for wg, b in ((w1g, b1), (w2g, b2)):
        hb = h.astype(jnp.bfloat16)
        m = jnp.dot(ones_row, (h * h).astype(jnp.bfloat16),
                    preferred_element_type=jnp.float32) * (1.0 / H)
        rs = lax.rsqrt(m + 1e-6)
        y = jnp.maximum(
            jnp.dot(wg[...], hb, preferred_element_type=jnp.float32) * rs
            + b[...],
            0.0)
        h = h + y
    return h


def _e2t_body(ec_ref, cnt_ref, lagr_ref,
              w0t, b0, w1g, b1, w2g, b2, wout, bout, e_ref):
    w0 = w0t[...]
    h = (w0[:, 0:1] * ec_ref[...] + w0[:, 1:2] * cnt_ref[...]
         + w0[:, 2:3] * lagr_ref[...] + b0[...])
    h = _mlp_t(h, w1g, b1, w2g, b2)
    logits = (jnp.dot(wout[...], h.astype(jnp.bfloat16),
                      preferred_element_type=jnp.float32) + bout[...])
    e = jnp.exp(jnp.clip(logits, -60.0, 60.0))
    col = (pl.program_id(0) * RB_E2T
           + lax.broadcasted_iota(jnp.int32, (1, RB_E2T), 1))
    e_ref[...] = jnp.where(col < N3, e, 0.0)


def _t2e_body(e12, e13, e23, a12, a13, a23, p12, p13, p23, q12, q13, q23,
              t12, t13, t23,
              w0t, b0, w1g, b1, w2g, b2, wout, bout,
              t12_o, t13_o, t23_o, d12_o, d13_o, d23_o):
    tu = []
    for e_r, a_r, p_r, q_r, t_r in ((e12, a12, p12, q12, t12),
                                    (e13, a13, p13, q13, t13),
                                    (e23, a23, p23, q23, t23)):
        w = e_r[...] / (p_r[...] + q_r[...])
        tu.append(t_r[...] + a_r[...] * w)
    w0 = w0t[...]
    h = (w0[:, 0:1] * tu[0] + w0[:, 1:2] * tu[1] + w0[:, 2:3] * tu[2] + b0[...])
    h = _mlp_t(h, w1g, b1, w2g, b2)
    delta = (jnp.dot(wout[...], h.astype(jnp.bfloat16),
                     preferred_element_type=jnp.float32)
             + bout[...])
    for k, (tu_k, to_ref, d_ref) in enumerate(
            ((tu[0], t12_o, d12_o), (tu[1], t13_o, d13_o), (tu[2], t23_o, d23_o))):
        d_k = delta[k:k + 1, :]
        d_ref[...] = d_k
        to_ref[...] = tu_k - d_k


def _combine_body(ec_ref, cnt_ref, p_ref, q_ref, out_ref):
    out_ref[...] = (jnp.where(cnt_ref[...] > 0, 0.0, ec_ref[...])
                    + p_ref[...] + q_ref[...])


def _row_spec(rb):
    return pl.BlockSpec((1, rb), lambda i: (0, i))


def _third_spec(rb, k):
    nb = T // rb
    return pl.BlockSpec((1, rb), lambda i, _k=k, _nb=nb: (0, _k * _nb + i))


def _full_spec(shape):
    return pl.BlockSpec(shape, lambda i: (0,) * len(shape))


RB_E2T = 4096
RB_T2E = 6400


def _prep_mlp_weights(W0, b0, g1, W1, b1, g2, W2, b2, Wout, bout):
    bf = jnp.bfloat16
    return (W0.T, b0.reshape(H, 1),
            (W1.T * g1[None, :]).astype(bf), b1.reshape(H, 1),
            (W2.T * g2[None, :]).astype(bf), b2.reshape(H, 1),
            Wout.T.astype(bf), bout.reshape(-1, 1))


def _weight_specs(ws):
    return [_full_spec(w.shape) for w in ws]


@jax.jit
def _run(edge_costs, t12_costs, t13_costs, t23_costs,
         tri_corr_12, tri_corr_13, tri_corr_23, edge_counter,
         e2t_w, t2e_w):
    pad_idx = (jnp.arange(NP - N3, dtype=jnp.int32) * 641) % E
    idx3 = jnp.concatenate(
        [tri_corr_12, tri_corr_13, tri_corr_23, pad_idx]).reshape(NW, NCH, CH)
    cnt_f = edge_counter.astype(jnp.float32)

    ecg3, cntg3 = _gather2(idx3, edge_costs, cnt_f)
    ec_g = ecg3.reshape(1, NP)
    cnt_g = cntg3.reshape(1, NP)
    lagr = jnp.concatenate(
        [t12_costs, t13_costs, t23_costs,
         jnp.zeros((NP - N3,), jnp.float32)]).reshape(1, NP)

    e2t_ws = _prep_mlp_weights(*e2t_w)
    e_row = pl.pallas_call(
        _e2t_body,
        grid=(NP // RB_E2T,),
        in_specs=[_row_spec(RB_E2T)] * 3 + _weight_specs(e2t_ws),
        out_specs=_row_spec(RB_E2T),
        out_shape=jax.ShapeDtypeStruct((1, NP), jnp.float32),
    )(ec_g, cnt_g, lagr, *e2t_ws)
    dpart = _scatter_add_kernel()(
        idx3, e_row.reshape(NW, NCH, CH)).reshape(NC, E)

    dg03, dg13 = _gather2(idx3, dpart[0], dpart[1])
    dg0 = dg03.reshape(1, NP)
    dg1 = dg13.reshape(1, NP)

    t2e_ws = _prep_mlp_weights(*t2e_w)
    thirds = lambda: [_third_spec(RB_T2E, k) for k in range(3)]
    outs = pl.pallas_call(
        _t2e_body,
        grid=(T // RB_T2E,),
        in_specs=(thirds() + thirds() + thirds() + thirds()
                  + [_row_spec(RB_T2E)] * 3 + _weight_specs(t2e_ws)),
        out_specs=[_row_spec(RB_T2E)] * 6,
        out_shape=[jax.ShapeDtypeStruct((1, T), jnp.float32)] * 6,
    )(e_row, e_row, e_row, ec_g, ec_g, ec_g, dg0, dg0, dg0, dg1, dg1, dg1,
      t12_costs.reshape(1, T), t13_costs.reshape(1, T), t23_costs.reshape(1, T),
      *t2e_ws)
    t12_o, t13_o, t23_o, d12, d13, d23 = outs

    dvals = jnp.concatenate(
        [d12.reshape(T), d13.reshape(T), d23.reshape(T),
         jnp.zeros((NP - N3,), jnp.float32)]).reshape(NW, NCH, CH)
    eupart = _scatter_add_kernel()(idx3, dvals)
    ec_o = _combine_sc_kernel()(eupart, edge_costs, edge_counter)

    return (ec_o, t12_o.reshape(T), t13_o.reshape(T), t23_o.reshape(T))


def kernel(edge_costs, t12_costs, t13_costs, t23_costs,
           tri_corr_12, tri_corr_13, tri_corr_23, edge_counter,
           e2t_W0, e2t_b0, e2t_g1, e2t_W1, e2t_b1, e2t_g2, e2t_W2, e2t_b2,
           e2t_Wout, e2t_bout,
           t2e_W0, t2e_b0, t2e_g1, t2e_W1, t2e_b1, t2e_g2, t2e_W2, t2e_b2,
           t2e_Wout, t2e_bout):
    e2t_w = (e2t_W0, e2t_b0, e2t_g1, e2t_W1, e2t_b1, e2t_g2, e2t_W2, e2t_b2,
             e2t_Wout, e2t_bout)
    t2e_w = (t2e_W0, t2e_b0, t2e_g1, t2e_W1, t2e_b1, t2e_g2, t2e_W2, t2e_b2,
             t2e_Wout, t2e_bout)
    return _run(edge_costs, t12_costs, t13_costs, t23_costs,
                tri_corr_12, tri_corr_13, tri_corr_23, edge_counter,
                e2t_w, t2e_w)


# speedup vs baseline: 1.1370x; 1.1370x over previous
import functools

import jax
import jax.numpy as jnp
from jax import lax
from jax.experimental import pallas as pl
from jax.experimental.pallas import tpu as pltpu
from jax.experimental.pallas import tpu_sc as plsc

E = 320000
T = 160000
N3 = 3 * T
H = 64

NC = 2
NS = 16
NW = NC * NS
CH = 128
NCH = 118
NP = NW * NCH * CH
ESH = E // NS
EW = E // NW


@functools.lru_cache(maxsize=None)
def _mesh():
    return plsc.VectorSubcoreMesh(
        core_axis_name="c", subcore_axis_name="s", num_cores=NC, num_subcores=NS)


def _zero_vmem(z_v, n):
    @pl.loop(0, n // 16)
    def _(k):
        z_v[pl.ds(k * 16, 16)] = jnp.zeros((16,), jnp.float32)


def _gather2_body(idx_hbm, ta_hbm, tb_hbm, outa_hbm, outb_hbm,
                  idx_v, a_v, b_v, s_v, ta_sh, tb_sh, sem_i, sem_a, sem_b):
    cid = lax.axis_index("c")
    sid = lax.axis_index("s")
    wid = sid * NC + cid
    sl = pl.ds(sid * ESH, ESH)

    idx_cp = pltpu.async_copy(idx_hbm.at[wid], idx_v, sem_i)
    pltpu.async_copy(ta_hbm.at[sl], s_v, sem_a).wait()
    pltpu.async_copy(s_v, ta_sh.at[sl], sem_a).wait()
    pltpu.async_copy(tb_hbm.at[sl], s_v, sem_b).wait()
    pltpu.async_copy(s_v, tb_sh.at[sl], sem_b).wait()
    idx_cp.wait()
    plsc.subcore_barrier()

    def start(j):
        pltpu.async_copy(ta_sh.at[idx_v.at[j]], a_v.at[j], sem_a)
        pltpu.async_copy(tb_sh.at[idx_v.at[j]], b_v.at[j], sem_b)

    def drain(j):
        pltpu.make_async_copy(ta_sh.at[idx_v.at[j]], a_v.at[j], sem_a).wait()
        pltpu.make_async_copy(tb_sh.at[idx_v.at[j]], b_v.at[j], sem_b).wait()

    start(0)

    @pl.loop(0, NCH - 1)
    def _(j):
        start(j + 1)
        drain(j)

    drain(NCH - 1)
    pltpu.async_copy(a_v, outa_hbm.at[wid], sem_a).wait()
    pltpu.async_copy(b_v, outb_hbm.at[wid], sem_b).wait()


@functools.lru_cache(maxsize=None)
def _gather2_kernel():
    return pl.kernel(
        _gather2_body,
        out_type=[jax.ShapeDtypeStruct((NW, NCH, CH), jnp.float32)] * 2,
        mesh=_mesh(),
        scratch_types=[
            pltpu.VMEM((NCH, CH), jnp.int32),
            pltpu.VMEM((NCH, CH), jnp.float32),
            pltpu.VMEM((NCH, CH), jnp.float32),
            pltpu.VMEM((ESH,), jnp.float32),
            pltpu.VMEM_SHARED((E,), jnp.float32),
            pltpu.VMEM_SHARED((E,), jnp.float32),
            pltpu.SemaphoreType.DMA,
            pltpu.SemaphoreType.DMA,
            pltpu.SemaphoreType.DMA,
        ],
    )


def _gather2(idx3, ta, tb):
    return _gather2_kernel()(idx3, ta, tb)


def _scatter_body(idx_hbm, val_hbm, out_hbm, idx_v, val_v, z_v, acc_sh, sem):
    cid = lax.axis_index("c")
    sid = lax.axis_index("s")
    wid = sid * NC + cid

    _zero_vmem(z_v, ESH)
    pltpu.async_copy(z_v, acc_sh.at[pl.ds(sid * ESH, ESH)], sem).wait()
    pltpu.async_copy(idx_hbm.at[wid], idx_v, sem).wait()
    pltpu.async_copy(val_hbm.at[wid], val_v, sem).wait()
    plsc.subcore_barrier()

    @pl.loop(0, NCH)
    def _(j):
        pltpu.sync_copy(val_v.at[j], acc_sh.at[idx_v.at[j]], add=True)

    plsc.subcore_barrier()
    pltpu.async_copy(acc_sh.at[pl.ds(sid * ESH, ESH)], z_v, sem).wait()
    pltpu.async_copy(
        z_v, out_hbm.at[pl.ds(cid * E + sid * ESH, ESH)], sem).wait()


@functools.lru_cache(maxsize=None)
def _scatter_add_kernel():
    return pl.kernel(
        _scatter_body,
        out_type=jax.ShapeDtypeStruct((NC * E,), jnp.float32),
        mesh=_mesh(),
        scratch_types=[
            pltpu.VMEM((NCH, CH), jnp.int32),
            pltpu.VMEM((NCH, CH), jnp.float32),
            pltpu.VMEM((ESH,), jnp.float32),
            pltpu.VMEM_SHARED((E,), jnp.float32),
            pltpu.SemaphoreType.DMA,
        ],
    )


def _combine_sc_body(p_hbm, ec_hbm, cnt_hbm, out_hbm,
                     p0_v, p1_v, ec_v, cnt_v, sem):
    cid = lax.axis_index("c")
    sid = lax.axis_index("s")
    wid = sid * NC + cid
    sl = pl.ds(wid * EW, EW)

    cp0 = pltpu.async_copy(p_hbm.at[pl.ds(wid * EW, EW)], p0_v, sem)
    cp1 = pltpu.async_copy(p_hbm.at[pl.ds(E + wid * EW, EW)], p1_v, sem)
    cpe = pltpu.async_copy(ec_hbm.at[sl], ec_v, sem)
    cpc = pltpu.async_copy(cnt_hbm.at[sl], cnt_v, sem)
    cp0.wait()
    cp1.wait()
    cpe.wait()
    cpc.wait()

    @pl.loop(0, EW // 16)
    def _(k):
        s16 = pl.ds(k * 16, 16)
        ec_v[s16] = (jnp.where(cnt_v[s16] > 0, 0.0, ec_v[s16])
                     + p0_v[s16] + p1_v[s16])

    pltpu.async_copy(ec_v, out_hbm.at[sl], sem).wait()


@functools.lru_cache(maxsize=None)
def _combine_sc_kernel():
    return pl.kernel(
        _combine_sc_body,
        out_type=jax.ShapeDtypeStruct((E,), jnp.float32),
        mesh=_mesh(),
        scratch_types=[
            pltpu.VMEM((EW,), jnp.float32),
            pltpu.VMEM((EW,), jnp.float32),
            pltpu.VMEM((EW,), jnp.float32),
            pltpu.VMEM((EW,), jnp.int32),
            pltpu.SemaphoreType.DMA,
        ],
    )


def _mlp_t(h, w1g, b1, w2g, b2):
    ones_row = jnp.ones((1, H), jnp.float32)
    for wg, b in ((w1g, b1), (w2g, b2)):
        m = jnp.dot(ones_row, h * h, preferred_element_type=jnp.float32) * (1.0 / H)
        rs = lax.rsqrt(m + 1e-6)
        y = jnp.maximum(
            jnp.dot(wg[...], h, preferred_element_type=jnp.float32) * rs + b[...],
            0.0)
        h = h + y
    return h


def _e2t_body(ec_ref, cnt_ref, lagr_ref,
              w0t, b0, w1g, b1, w2g, b2, wout, bout, e_ref):
    w0 = w0t[...]
    h = (w0[:, 0:1] * ec_ref[...] + w0[:, 1:2] * cnt_ref[...]
         + w0[:, 2:3] * lagr_ref[...] + b0[...])
    h = _mlp_t(h, w1g, b1, w2g, b2)
    logits = (jnp.dot(wout[...], h, preferred_element_type=jnp.float32)
              + bout[...])
    e = jnp.exp(jnp.clip(logits, -60.0, 60.0))
    col = (pl.program_id(0) * RB_E2T
           + lax.broadcasted_iota(jnp.int32, (1, RB_E2T), 1))
    e_ref[...] = jnp.where(col < N3, e, 0.0)


def _t2e_body(e12, e13, e23, a12, a13, a23, p12, p13, p23, q12, q13, q23,
              t12, t13, t23,
              w0t, b0, w1g, b1, w2g, b2, wout, bout,
              t12_o, t13_o, t23_o, d12_o, d13_o, d23_o):
    tu = []
    for e_r, a_r, p_r, q_r, t_r in ((e12, a12, p12, q12, t12),
                                    (e13, a13, p13, q13, t13),
                                    (e23, a23, p23, q23, t23)):
        w = e_r[...] / (p_r[...] + q_r[...])
        tu.append(t_r[...] + a_r[...] * w)
    w0 = w0t[...]
    h = (w0[:, 0:1] * tu[0] + w0[:, 1:2] * tu[1] + w0[:, 2:3] * tu[2] + b0[...])
    h = _mlp_t(h, w1g, b1, w2g, b2)
    delta = (jnp.dot(wout[...], h, preferred_element_type=jnp.float32)
             + bout[...])
    for k, (tu_k, to_ref, d_ref) in enumerate(
            ((tu[0], t12_o, d12_o), (tu[1], t13_o, d13_o), (tu[2], t23_o, d23_o))):
        d_k = delta[k:k + 1, :]
        d_ref[...] = d_k
        to_ref[...] = tu_k - d_k


def _row_spec(rb):
    return pl.BlockSpec((1, rb), lambda i: (0, i))


def _third_spec(rb, k):
    nb = T // rb
    return pl.BlockSpec((1, rb), lambda i, _k=k, _nb=nb: (0, _k * _nb + i))


def _full_spec(shape):
    return pl.BlockSpec(shape, lambda i: (0,) * len(shape))


RB_E2T = 15104
RB_T2E = 16000


def _prep_mlp_weights(W0, b0, g1, W1, b1, g2, W2, b2, Wout, bout):
    return (W0.T, b0.reshape(H, 1), W1.T * g1[None, :], b1.reshape(H, 1),
            W2.T * g2[None, :], b2.reshape(H, 1), Wout.T,
            bout.reshape(-1, 1))


def _weight_specs(ws):
    return [_full_spec(w.shape) for w in ws]


@jax.jit
def _run(edge_costs, t12_costs, t13_costs, t23_costs,
         tri_corr_12, tri_corr_13, tri_corr_23, edge_counter,
         e2t_w, t2e_w):
    pad_idx = (jnp.arange(NP - N3, dtype=jnp.int32) * 641) % E
    idx3 = jnp.concatenate(
        [tri_corr_12, tri_corr_13, tri_corr_23, pad_idx]).reshape(NW, NCH, CH)
    cnt_f = edge_counter.astype(jnp.float32)

    ecg3, cntg3 = _gather2(idx3, edge_costs, cnt_f)
    ec_g = ecg3.reshape(1, NP)
    cnt_g = cntg3.reshape(1, NP)
    lagr = jnp.concatenate(
        [t12_costs, t13_costs, t23_costs,
         jnp.zeros((NP - N3,), jnp.float32)]).reshape(1, NP)

    e2t_ws = _prep_mlp_weights(*e2t_w)
    e_row = pl.pallas_call(
        _e2t_body,
        grid=(NP // RB_E2T,),
        in_specs=[_row_spec(RB_E2T)] * 3 + _weight_specs(e2t_ws),
        out_specs=_row_spec(RB_E2T),
        out_shape=jax.ShapeDtypeStruct((1, NP), jnp.float32),
    )(ec_g, cnt_g, lagr, *e2t_ws)
    dpart = _scatter_add_kernel()(
        idx3, e_row.reshape(NW, NCH, CH)).reshape(NC, E)

    dg03, dg13 = _gather2(idx3, dpart[0], dpart[1])
    dg0 = dg03.reshape(1, NP)
    dg1 = dg13.reshape(1, NP)

    t2e_ws = _prep_mlp_weights(*t2e_w)
    thirds = lambda: [_third_spec(RB_T2E, k) for k in range(3)]
    outs = pl.pallas_call(
        _t2e_body,
        grid=(T // RB_T2E,),
        in_specs=(thirds() + thirds() + thirds() + thirds() + thirds()
                  + _weight_specs(t2e_ws)),
        out_specs=[_row_spec(RB_T2E)] * 6,
        out_shape=[jax.ShapeDtypeStruct((1, T), jnp.float32)] * 6,
    )(e_row, e_row, e_row, ec_g, ec_g, ec_g, dg0, dg0, dg0, dg1, dg1, dg1,
      lagr, lagr, lagr, *t2e_ws)
    t12_o, t13_o, t23_o, d12, d13, d23 = outs

    dvals = jnp.concatenate(
        [d12.reshape(T), d13.reshape(T), d23.reshape(T),
         jnp.zeros((NP - N3,), jnp.float32)]).reshape(NW, NCH, CH)
    eupart = _scatter_add_kernel()(idx3, dvals)
    ec_o = _combine_sc_kernel()(eupart, edge_costs, edge_counter)

    return (ec_o, t12_o.reshape(T), t13_o.reshape(T), t23_o.reshape(T))


def kernel(edge_costs, t12_costs, t13_costs, t23_costs,
           tri_corr_12, tri_corr_13, tri_corr_23, edge_counter,
           e2t_W0, e2t_b0, e2t_g1, e2t_W1, e2t_b1, e2t_g2, e2t_W2, e2t_b2,
           e2t_Wout, e2t_bout,
           t2e_W0, t2e_b0, t2e_g1, t2e_W1, t2e_b1, t2e_g2, t2e_W2, t2e_b2,
           t2e_Wout, t2e_bout):
    e2t_w = (e2t_W0, e2t_b0, e2t_g1, e2t_W1, e2t_b1, e2t_g2, e2t_W2, e2t_b2,
             e2t_Wout, e2t_bout)
    t2e_w = (t2e_W0, t2e_b0, t2e_g1, t2e_W1, t2e_b1, t2e_g2, t2e_W2, t2e_b2,
             t2e_Wout, t2e_bout)
    return _run(edge_costs, t12_costs, t13_costs, t23_costs,
                tri_corr_12, tri_corr_13, tri_corr_23, edge_counter,
                e2t_w, t2e_w)
